# Initial kernel scaffold; baseline (speedup 1.0000x reference)
import jax
import jax.numpy as jnp
from jax.experimental import pallas as pl


def kernel(x_hru, x_ws, x_gw, x_channel, ei_climate_src, ei_climate_dst, ei_swgw_src, ei_swgw_dst, ei_gwsw_src, ei_gwsw_dst, ei_sw_src, ei_sw_dst, W_climate, b_climate, W_swgw, b_swgw, W_gwsw, b_gwsw, W_sw, b_sw, W_ih0, W_hh0, b_ih0, b_hh0, W_ih1, W_hh1, b_ih1, b_hh1, fc1_w, fc1_b, fc2_w, fc2_b):
    raise NotImplementedError("write your pallas kernel here")


# speedup vs baseline: 9.7121x; 9.7121x over previous
import jax
import jax.numpy as jnp
from jax import lax
from jax.experimental import pallas as pl
from jax.experimental.pallas import tpu as pltpu
from jax.experimental.pallas import tpu_sc as plsc

N_CH = 5000
N_PAD = 5120
N_SEND = 50000
N_SEND_PAD = 50048
HID = 128
T_STEPS = 5000
E = 150000
NT = 16
BLK = 128
EB = 76
E_TILE = EB * BLK
E_PAD = NT * E_TILE
GRP = 4
NGRP = EB // GRP


def _deg_body(idx_send, idx_recv, zeros_big, ones_rows,
              hist_send, hist_recv,
              spm_s, spm_r, zbuf, sidx_v, ridx_v, ones_v, sem_s, sem_r):
    cid = lax.axis_index("c")
    sid = lax.axis_index("s")
    rb = N_SEND_PAD // NT
    rc = N_PAD // NT
    pltpu.sync_copy(zeros_big, zbuf)
    pltpu.sync_copy(zbuf, spm_s.at[pl.ds(sid * rb, rb)])
    pltpu.sync_copy(zbuf.at[pl.ds(0, rc)], spm_r.at[pl.ds(sid * rc, rc)])
    pltpu.sync_copy(ones_rows, ones_v)
    pltpu.sync_copy(idx_send.at[cid, sid], sidx_v)
    pltpu.sync_copy(idx_recv.at[cid, sid], ridx_v)
    plsc.subcore_barrier()

    def grp(g, carry):
        hs = []
        for b in range(GRP):
            j = g * GRP + b
            hs.append(pltpu.async_copy(ones_v, spm_s.at[sidx_v.at[j]],
                                       sem_s, add=True))
            hs.append(pltpu.async_copy(ones_v, spm_r.at[ridx_v.at[j]],
                                       sem_r, add=True))
        for h in hs:
            h.wait()
        return carry

    lax.fori_loop(0, NGRP, grp, 0)
    plsc.subcore_barrier()
    pltpu.sync_copy(spm_s.at[pl.ds(sid * rb, rb)], zbuf)
    pltpu.sync_copy(zbuf, hist_send.at[cid, pl.ds(sid * rb, rb)])
    pltpu.sync_copy(spm_r.at[pl.ds(sid * rc, rc)], zbuf.at[pl.ds(0, rc)])
    pltpu.sync_copy(zbuf.at[pl.ds(0, rc)],
                    hist_recv.at[cid, pl.ds(sid * rc, rc)])


def _run_degrees(idx_send2, idx_recv2):
    mesh = plsc.VectorSubcoreMesh(core_axis_name="c", subcore_axis_name="s")
    ones_rows = jnp.zeros((BLK, 16), jnp.float32).at[:, 0].set(1.0)
    rb = N_SEND_PAD // NT
    return pl.kernel(
        _deg_body,
        out_type=[jax.ShapeDtypeStruct((2, N_SEND_PAD, 16), jnp.float32),
                  jax.ShapeDtypeStruct((2, N_PAD, 16), jnp.float32)],
        mesh=mesh,
        compiler_params=pltpu.CompilerParams(use_tc_tiling_on_sc=False),
        scratch_types=[
            pltpu.VMEM_SHARED((N_SEND_PAD, 16), jnp.float32),
            pltpu.VMEM_SHARED((N_PAD, 16), jnp.float32),
            pltpu.VMEM((rb, 16), jnp.float32),
            pltpu.VMEM((EB, BLK), jnp.int32),
            pltpu.VMEM((EB, BLK), jnp.int32),
            pltpu.VMEM((BLK, 16), jnp.float32),
            pltpu.SemaphoreType.DMA,
            pltpu.SemaphoreType.DMA,
        ],
    )(idx_send2, idx_recv2, jnp.zeros((rb, 16), jnp.float32), ones_rows)


def _agg_body(h_a, h_b, idx_send, idx_recv, zeros_acc,
              acc_a, acc_b,
              spm_acc, sidx_v, ridx_v,
              buf0, buf1, buf2, buf3,
              sg0, sg1, sg2, sg3, ss0, ss1, ss2, ss3):
    cid = lax.axis_index("c")
    sid = lax.axis_index("s")
    rc = N_PAD // NT
    bufs = (buf0, buf1, buf2, buf3)
    sgs = (sg0, sg1, sg2, sg3)
    sss = (ss0, ss1, ss2, ss3)
    pltpu.sync_copy(zeros_acc, buf0.at[pl.ds(0, 64)])
    for k in range(rc // 64):
        pltpu.sync_copy(buf0.at[pl.ds(0, 64)],
                        spm_acc.at[pl.ds(sid * rc + k * 64, 64)])
    pltpu.sync_copy(idx_send.at[cid, sid], sidx_v)
    pltpu.sync_copy(idx_recv.at[cid, sid], ridx_v)
    plsc.subcore_barrier()

    def run(h_table):
        for b in range(GRP):
            pltpu.async_copy(h_table.at[sidx_v.at[b]], bufs[b], sgs[b])

        def grp(g, carry):
            for b in range(GRP):
                j = g * GRP + b
                pltpu.make_async_copy(h_table.at[sidx_v.at[0]],
                                      bufs[b], sgs[b]).wait()
                pltpu.async_copy(bufs[b], spm_acc.at[ridx_v.at[j]],
                                 sss[b], add=True)
            for b in range(GRP):
                pltpu.make_async_copy(bufs[b], spm_acc.at[ridx_v.at[0]],
                                      sss[b]).wait()

            @pl.when(g < NGRP - 1)
            def _():
                for b in range(GRP):
                    jn = (g + 1) * GRP + b
                    pltpu.async_copy(h_table.at[sidx_v.at[jn]],
                                     bufs[b], sgs[b])
            return carry

        lax.fori_loop(0, NGRP, grp, 0)

    @pl.when(cid == 0)
    def _():
        run(h_a)

    @pl.when(cid == 1)
    def _():
        run(h_b)

    plsc.subcore_barrier()

    def export(out_ref):
        for k in range(rc // 64):
            pltpu.sync_copy(spm_acc.at[pl.ds(sid * rc + k * 64, 64)],
                            buf0.at[pl.ds(0, 64)])
            pltpu.sync_copy(buf0.at[pl.ds(0, 64)],
                            out_ref.at[pl.ds(sid * rc + k * 64, 64)])

    @pl.when(cid == 0)
    def _():
        export(acc_a)

    @pl.when(cid == 1)
    def _():
        export(acc_b)


def _run_agg(h_a, h_b, idx_send2, idx_recv2):
    mesh = plsc.VectorSubcoreMesh(core_axis_name="c", subcore_axis_name="s")
    sems = [pltpu.SemaphoreType.DMA] * (2 * GRP)
    return pl.kernel(
        _agg_body,
        out_type=[jax.ShapeDtypeStruct((N_PAD, HID), jnp.float32),
                  jax.ShapeDtypeStruct((N_PAD, HID), jnp.float32)],
        mesh=mesh,
        compiler_params=pltpu.CompilerParams(use_tc_tiling_on_sc=False),
        scratch_types=[
            pltpu.VMEM_SHARED((N_PAD, HID), jnp.float32),
            pltpu.VMEM((EB, BLK), jnp.int32),
            pltpu.VMEM((EB, BLK), jnp.int32),
            pltpu.VMEM((BLK, HID), jnp.float32),
            pltpu.VMEM((BLK, HID), jnp.float32),
            pltpu.VMEM((BLK, HID), jnp.float32),
            pltpu.VMEM((BLK, HID), jnp.float32),
        ] + sems,
    )(h_a, h_b, idx_send2, idx_recv2,
      jnp.zeros((64, HID), jnp.float32))


def _h_kernel(x_ref, w_ref, hist_ref, o_ref):
    rs = lax.rsqrt(jnp.maximum(hist_ref[...][:, 0:1], 1.0))
    o_ref[...] = jnp.dot(x_ref[...], w_ref[...],
                         preferred_element_type=jnp.float32) * rs


def _scaled_h(x_pad, w, hist):
    blk = N_SEND_PAD // 8
    return pl.pallas_call(
        _h_kernel,
        grid=(8,),
        in_specs=[pl.BlockSpec((blk, HID), lambda i: (i, 0)),
                  pl.BlockSpec((HID, HID), lambda i: (0, 0)),
                  pl.BlockSpec((blk, 16), lambda i: (i, 0))],
        out_specs=pl.BlockSpec((blk, HID), lambda i: (i, 0)),
        out_shape=jax.ShapeDtypeStruct((N_SEND_PAD, HID), jnp.float32),
    )(x_pad, w, hist)


def _head_kernel(acc_a_ref, acc_b_ref, hist_a_ref, hist_b_ref,
                 b_a_ref, b_b_ref,
                 wih0t_ref, bih0_ref, bhh0_ref,
                 wcat1_ref, whh1t_ref, bih1_ref, bhh1_ref,
                 out_ref, gx_ref):
    rs_a = lax.rsqrt(jnp.maximum(hist_a_ref[...][:, 0:1], 1.0))
    rs_b = lax.rsqrt(jnp.maximum(hist_b_ref[...][:, 0:1], 1.0))
    h_ch = ((acc_a_ref[...] * rs_a + b_a_ref[...])
            + (acc_b_ref[...] * rs_b + b_b_ref[...]))
    gx_ref[...] = jnp.dot(h_ch, wih0t_ref[...],
                          preferred_element_type=jnp.float32) + bih0_ref[...]

    wcat1 = wcat1_ref[...]
    whh1t = whh1t_ref[...]
    bhh0 = bhh0_ref[...]
    bih1 = bih1_ref[...]
    bhh1 = bhh1_ref[...]

    def cell(g, c):
        i = jax.nn.sigmoid(g[:, :HID])
        f = jax.nn.sigmoid(g[:, HID:2 * HID])
        gg = jnp.tanh(g[:, 2 * HID:3 * HID])
        o = jax.nn.sigmoid(g[:, 3 * HID:])
        c2 = f * c + i * gg
        return o * jnp.tanh(c2), c2

    h1, c1 = cell(gx_ref[pl.ds(0, 1), :] + bhh0, jnp.zeros((1, HID), jnp.float32))
    h2 = jnp.zeros((1, HID), jnp.float32)
    c2 = jnp.zeros((1, HID), jnp.float32)

    def body(t, carry):
        h1, c1, h2, c2 = carry
        z1 = jnp.dot(h1, wcat1, preferred_element_type=jnp.float32)
        z2 = jnp.dot(h2, whh1t, preferred_element_type=jnp.float32)
        g1 = (gx_ref[pl.ds(t, 1), :] + z1[:, :4 * HID]) + bhh0
        h1n, c1n = cell(g1, c1)
        g2 = ((z1[:, 4 * HID:] + bih1) + z2) + bhh1
        h2n, c2n = cell(g2, c2)
        return h1n, c1n, h2n, c2n

    h1, c1, h2, c2 = lax.fori_loop(1, T_STEPS + 1, body, (h1, c1, h2, c2))
    out_ref[...] = h2


def _run_head(acc_a, acc_b, hist_a, hist_b, b_a, b_b,
              W_ih0, W_hh0, b_ih0, b_hh0, W_ih1, W_hh1, b_ih1, b_hh1,
              fc1_w, fc1_b, fc2_w, fc2_b):
    wcat1 = jnp.concatenate([W_hh0.T, W_ih1.T], axis=1)
    h2 = pl.pallas_call(
        _head_kernel,
        out_shape=jax.ShapeDtypeStruct((1, HID), jnp.float32),
        scratch_shapes=[pltpu.VMEM((N_PAD, 4 * HID), jnp.float32)],
    )(acc_a, acc_b, hist_a, hist_b, b_a[None, :], b_b[None, :],
      W_ih0.T, b_ih0[None, :], b_hh0[None, :],
      wcat1, W_hh1.T, b_ih1[None, :], b_hh1[None, :])
    last = h2[0]
    z = jnp.maximum(last @ fc1_w.T + fc1_b, 0.0)
    return (z @ fc2_w.T + fc2_b)[None, :]


def _prep_idx(idx, fill):
    pad = jnp.full((E_PAD - E,), fill, jnp.int32)
    return jnp.concatenate([idx, pad]).reshape(NT, EB, BLK)


def kernel(x_hru, x_ws, x_gw, x_channel,
           ei_climate_src, ei_climate_dst, ei_swgw_src, ei_swgw_dst,
           ei_gwsw_src, ei_gwsw_dst, ei_sw_src, ei_sw_dst,
           W_climate, b_climate, W_swgw, b_swgw, W_gwsw, b_gwsw, W_sw, b_sw,
           W_ih0, W_hh0, b_ih0, b_hh0, W_ih1, W_hh1, b_ih1, b_hh1,
           fc1_w, fc1_b, fc2_w, fc2_b):
    idx_send2 = jnp.stack([_prep_idx(ei_gwsw_dst, N_SEND),
                           _prep_idx(ei_sw_dst, N_SEND)])
    idx_recv2 = jnp.stack([_prep_idx(ei_gwsw_src, N_CH),
                           _prep_idx(ei_sw_src, N_CH)])

    hist_send, hist_recv = _run_degrees(idx_send2, idx_recv2)

    x_a = jnp.pad(x_gw, ((0, N_SEND_PAD - N_SEND), (0, 0)))
    x_b = jnp.pad(x_hru, ((0, N_SEND_PAD - N_SEND), (0, 0)))
    h_a = _scaled_h(x_a, W_gwsw, hist_send[0])
    h_b = _scaled_h(x_b, W_sw, hist_send[1])

    acc_a, acc_b = _run_agg(h_a, h_b, idx_send2, idx_recv2)

    return _run_head(acc_a, acc_b, hist_recv[0], hist_recv[1], b_gwsw, b_sw,
                     W_ih0, W_hh0, b_ih0, b_hh0,
                     W_ih1, W_hh1, b_ih1, b_hh1,
                     fc1_w, fc1_b, fc2_w, fc2_b)


# speedup vs baseline: 10.5179x; 1.0830x over previous
import jax
import jax.numpy as jnp
from jax import lax
from jax.experimental import pallas as pl
from jax.experimental.pallas import tpu as pltpu
from jax.experimental.pallas import tpu_sc as plsc

N_CH = 5000
N_PAD = 5120
N_SEND = 50000
N_SEND_PAD = 50048
HID = 128
T_STEPS = 5000
E = 150000
NT = 16
BLK = 128
EB = 76
E_TILE = EB * BLK
E_PAD = NT * E_TILE
GRP = 4
NGRP = EB // GRP


def _deg_body(idx_send, idx_recv, zeros_big, ones_rows,
              hist_send, hist_recv,
              spm_s, spm_r, zbuf, sidx_v, ridx_v, ones_v, sem_s, sem_r):
    cid = lax.axis_index("c")
    sid = lax.axis_index("s")
    rb = N_SEND_PAD // NT
    rc = N_PAD // NT
    pltpu.sync_copy(zeros_big, zbuf)
    pltpu.sync_copy(zbuf, spm_s.at[pl.ds(sid * rb, rb)])
    pltpu.sync_copy(zbuf.at[pl.ds(0, rc)], spm_r.at[pl.ds(sid * rc, rc)])
    pltpu.sync_copy(ones_rows, ones_v)
    pltpu.sync_copy(idx_send.at[cid, sid], sidx_v)
    pltpu.sync_copy(idx_recv.at[cid, sid], ridx_v)
    plsc.subcore_barrier()

    def grp(g, carry):
        hs = []
        for b in range(GRP):
            j = g * GRP + b
            hs.append(pltpu.async_copy(ones_v, spm_s.at[sidx_v.at[j]],
                                       sem_s, add=True))
            hs.append(pltpu.async_copy(ones_v, spm_r.at[ridx_v.at[j]],
                                       sem_r, add=True))
        for h in hs:
            h.wait()
        return carry

    lax.fori_loop(0, NGRP, grp, 0)
    plsc.subcore_barrier()
    pltpu.sync_copy(spm_s.at[pl.ds(sid * rb, rb)], zbuf)
    pltpu.sync_copy(zbuf, hist_send.at[cid, pl.ds(sid * rb, rb)])
    pltpu.sync_copy(spm_r.at[pl.ds(sid * rc, rc)], zbuf.at[pl.ds(0, rc)])
    pltpu.sync_copy(zbuf.at[pl.ds(0, rc)],
                    hist_recv.at[cid, pl.ds(sid * rc, rc)])


def _run_degrees(idx_send2, idx_recv2):
    mesh = plsc.VectorSubcoreMesh(core_axis_name="c", subcore_axis_name="s")
    ones_rows = jnp.zeros((BLK, 16), jnp.float32).at[:, 0].set(1.0)
    rb = N_SEND_PAD // NT
    return pl.kernel(
        _deg_body,
        out_type=[jax.ShapeDtypeStruct((2, N_SEND_PAD, 16), jnp.float32),
                  jax.ShapeDtypeStruct((2, N_PAD, 16), jnp.float32)],
        mesh=mesh,
        compiler_params=pltpu.CompilerParams(use_tc_tiling_on_sc=False),
        scratch_types=[
            pltpu.VMEM_SHARED((N_SEND_PAD, 16), jnp.float32),
            pltpu.VMEM_SHARED((N_PAD, 16), jnp.float32),
            pltpu.VMEM((rb, 16), jnp.float32),
            pltpu.VMEM((EB, BLK), jnp.int32),
            pltpu.VMEM((EB, BLK), jnp.int32),
            pltpu.VMEM((BLK, 16), jnp.float32),
            pltpu.SemaphoreType.DMA,
            pltpu.SemaphoreType.DMA,
        ],
    )(idx_send2, idx_recv2, jnp.zeros((rb, 16), jnp.float32), ones_rows)


def _agg_body(h_a, h_b, idx_send, idx_recv, zeros_acc,
              acc_a, acc_b,
              spm_acc, sidx_v, ridx_v,
              buf0, buf1, buf2, buf3,
              sg0, sg1, sg2, sg3, ss0, ss1, ss2, ss3):
    cid = lax.axis_index("c")
    sid = lax.axis_index("s")
    rc = N_PAD // NT
    bufs = (buf0, buf1, buf2, buf3)
    sgs = (sg0, sg1, sg2, sg3)
    sss = (ss0, ss1, ss2, ss3)
    pltpu.sync_copy(zeros_acc, buf0.at[pl.ds(0, 64)])
    for k in range(rc // 64):
        pltpu.sync_copy(buf0.at[pl.ds(0, 64)],
                        spm_acc.at[pl.ds(sid * rc + k * 64, 64)])
    pltpu.sync_copy(idx_send.at[cid, sid], sidx_v)
    pltpu.sync_copy(idx_recv.at[cid, sid], ridx_v)
    plsc.subcore_barrier()

    def run(h_table):
        for b in range(GRP):
            pltpu.async_copy(h_table.at[sidx_v.at[b]], bufs[b], sgs[b])

        def grp(g, carry):
            for b in range(GRP):
                j = g * GRP + b
                pltpu.make_async_copy(h_table.at[sidx_v.at[0]],
                                      bufs[b], sgs[b]).wait()
                pltpu.async_copy(bufs[b], spm_acc.at[ridx_v.at[j]],
                                 sss[b], add=True)
            for b in range(GRP):
                pltpu.make_async_copy(bufs[b], spm_acc.at[ridx_v.at[0]],
                                      sss[b]).wait()

            @pl.when(g < NGRP - 1)
            def _():
                for b in range(GRP):
                    jn = (g + 1) * GRP + b
                    pltpu.async_copy(h_table.at[sidx_v.at[jn]],
                                     bufs[b], sgs[b])
            return carry

        lax.fori_loop(0, NGRP, grp, 0)

    @pl.when(cid == 0)
    def _():
        run(h_a)

    @pl.when(cid == 1)
    def _():
        run(h_b)

    plsc.subcore_barrier()

    def export(out_ref):
        for k in range(rc // 64):
            pltpu.sync_copy(spm_acc.at[pl.ds(sid * rc + k * 64, 64)],
                            buf0.at[pl.ds(0, 64)])
            pltpu.sync_copy(buf0.at[pl.ds(0, 64)],
                            out_ref.at[pl.ds(sid * rc + k * 64, 64)])

    @pl.when(cid == 0)
    def _():
        export(acc_a)

    @pl.when(cid == 1)
    def _():
        export(acc_b)


def _run_agg(h_a, h_b, idx_send2, idx_recv2):
    mesh = plsc.VectorSubcoreMesh(core_axis_name="c", subcore_axis_name="s")
    sems = [pltpu.SemaphoreType.DMA] * (2 * GRP)
    return pl.kernel(
        _agg_body,
        out_type=[jax.ShapeDtypeStruct((N_PAD, HID), jnp.float32),
                  jax.ShapeDtypeStruct((N_PAD, HID), jnp.float32)],
        mesh=mesh,
        compiler_params=pltpu.CompilerParams(use_tc_tiling_on_sc=False),
        scratch_types=[
            pltpu.VMEM_SHARED((N_PAD, HID), jnp.float32),
            pltpu.VMEM((EB, BLK), jnp.int32),
            pltpu.VMEM((EB, BLK), jnp.int32),
            pltpu.VMEM((BLK, HID), jnp.float32),
            pltpu.VMEM((BLK, HID), jnp.float32),
            pltpu.VMEM((BLK, HID), jnp.float32),
            pltpu.VMEM((BLK, HID), jnp.float32),
        ] + sems,
    )(h_a, h_b, idx_send2, idx_recv2,
      jnp.zeros((64, HID), jnp.float32))


def _h_kernel(x_ref, w_ref, hist_ref, o_ref):
    rs = lax.rsqrt(jnp.maximum(hist_ref[...][:, 0:1], 1.0))
    o_ref[...] = jnp.dot(x_ref[...], w_ref[...],
                         preferred_element_type=jnp.float32) * rs


def _scaled_h(x_pad, w, hist):
    blk = N_SEND_PAD // 8
    return pl.pallas_call(
        _h_kernel,
        grid=(8,),
        in_specs=[pl.BlockSpec((blk, HID), lambda i: (i, 0)),
                  pl.BlockSpec((HID, HID), lambda i: (0, 0)),
                  pl.BlockSpec((blk, 16), lambda i: (i, 0))],
        out_specs=pl.BlockSpec((blk, HID), lambda i: (i, 0)),
        out_shape=jax.ShapeDtypeStruct((N_SEND_PAD, HID), jnp.float32),
    )(x_pad, w, hist)


def _head_kernel(acc_a_ref, acc_b_ref, hist_a_ref, hist_b_ref,
                 b_a_ref, b_b_ref,
                 wih0t_ref, bih0_ref, bhh0_ref,
                 wcat1_ref, whh1t_ref, bih1_ref, bhh1_ref,
                 out_ref, gx_ref):
    rs_a = lax.rsqrt(jnp.maximum(hist_a_ref[...][:, 0:1], 1.0))
    rs_b = lax.rsqrt(jnp.maximum(hist_b_ref[...][:, 0:1], 1.0))
    h_ch = ((acc_a_ref[...] * rs_a + b_a_ref[...])
            + (acc_b_ref[...] * rs_b + b_b_ref[...]))
    gx_ref[...] = jnp.dot(h_ch, wih0t_ref[...],
                          preferred_element_type=jnp.float32) + bih0_ref[...]

    wcat1 = wcat1_ref[...]
    whh1t = whh1t_ref[...]
    bhh0 = bhh0_ref[...]
    bih1 = bih1_ref[...]
    bhh1 = bhh1_ref[...]

    def cell(g, c):
        sg = jax.nn.sigmoid(g[:, :3 * HID])
        gg = jnp.tanh(g[:, 3 * HID:])
        i = sg[:, :HID]
        f = sg[:, HID:2 * HID]
        o = sg[:, 2 * HID:3 * HID]
        c2 = f * c + i * gg
        return o * jnp.tanh(c2), c2

    h1, c1 = cell(gx_ref[pl.ds(0, 1), :] + bhh0, jnp.zeros((1, HID), jnp.float32))
    h2 = jnp.zeros((1, HID), jnp.float32)
    c2 = jnp.zeros((1, HID), jnp.float32)

    def step(t, h1, c1, h2, c2):
        z1 = jnp.dot(h1, wcat1, preferred_element_type=jnp.float32)
        z2 = jnp.dot(h2, whh1t, preferred_element_type=jnp.float32)
        g1 = (gx_ref[pl.ds(t, 1), :] + z1[:, :4 * HID]) + bhh0
        h1n, c1n = cell(g1, c1)
        g2 = ((z1[:, 4 * HID:] + bih1) + z2) + bhh1
        h2n, c2n = cell(g2, c2)
        return h1n, c1n, h2n, c2n

    def body(tt, carry):
        t = 2 * tt + 1
        carry = step(t, *carry)
        carry = step(t + 1, *carry)
        return carry

    h1, c1, h2, c2 = lax.fori_loop(0, T_STEPS // 2, body, (h1, c1, h2, c2))
    out_ref[...] = h2


def _run_head(acc_a, acc_b, hist_a, hist_b, b_a, b_b,
              W_ih0, W_hh0, b_ih0, b_hh0, W_ih1, W_hh1, b_ih1, b_hh1,
              fc1_w, fc1_b, fc2_w, fc2_b):
    perm = jnp.concatenate([jnp.arange(0, 2 * HID), jnp.arange(3 * HID, 4 * HID),
                            jnp.arange(2 * HID, 3 * HID)])
    W_ih0 = W_ih0[perm]
    W_hh0 = W_hh0[perm]
    b_ih0 = b_ih0[perm]
    b_hh0 = b_hh0[perm]
    W_ih1 = W_ih1[perm]
    W_hh1 = W_hh1[perm]
    b_ih1 = b_ih1[perm]
    b_hh1 = b_hh1[perm]
    wcat1 = jnp.concatenate([W_hh0.T, W_ih1.T], axis=1)
    h2 = pl.pallas_call(
        _head_kernel,
        out_shape=jax.ShapeDtypeStruct((1, HID), jnp.float32),
        scratch_shapes=[pltpu.VMEM((N_PAD, 4 * HID), jnp.float32)],
    )(acc_a, acc_b, hist_a, hist_b, b_a[None, :], b_b[None, :],
      W_ih0.T, b_ih0[None, :], b_hh0[None, :],
      wcat1, W_hh1.T, b_ih1[None, :], b_hh1[None, :])
    last = h2[0]
    z = jnp.maximum(last @ fc1_w.T + fc1_b, 0.0)
    return (z @ fc2_w.T + fc2_b)[None, :]


def _prep_idx(idx, fill):
    pad = jnp.full((E_PAD - E,), fill, jnp.int32)
    return jnp.concatenate([idx, pad]).reshape(NT, EB, BLK)


def kernel(x_hru, x_ws, x_gw, x_channel,
           ei_climate_src, ei_climate_dst, ei_swgw_src, ei_swgw_dst,
           ei_gwsw_src, ei_gwsw_dst, ei_sw_src, ei_sw_dst,
           W_climate, b_climate, W_swgw, b_swgw, W_gwsw, b_gwsw, W_sw, b_sw,
           W_ih0, W_hh0, b_ih0, b_hh0, W_ih1, W_hh1, b_ih1, b_hh1,
           fc1_w, fc1_b, fc2_w, fc2_b):
    idx_send2 = jnp.stack([_prep_idx(ei_gwsw_dst, N_SEND),
                           _prep_idx(ei_sw_dst, N_SEND)])
    idx_recv2 = jnp.stack([_prep_idx(ei_gwsw_src, N_CH),
                           _prep_idx(ei_sw_src, N_CH)])

    hist_send, hist_recv = _run_degrees(idx_send2, idx_recv2)

    x_a = jnp.pad(x_gw, ((0, N_SEND_PAD - N_SEND), (0, 0)))
    x_b = jnp.pad(x_hru, ((0, N_SEND_PAD - N_SEND), (0, 0)))
    h_a = _scaled_h(x_a, W_gwsw, hist_send[0])
    h_b = _scaled_h(x_b, W_sw, hist_send[1])

    acc_a, acc_b = _run_agg(h_a, h_b, idx_send2, idx_recv2)

    return _run_head(acc_a, acc_b, hist_recv[0], hist_recv[1], b_gwsw, b_sw,
                     W_ih0, W_hh0, b_ih0, b_hh0,
                     W_ih1, W_hh1, b_ih1, b_hh1,
                     fc1_w, fc1_b, fc2_w, fc2_b)
